# baseline (speedup 1.0000x reference)
import functools

import jax
import jax.numpy as jnp
from jax import lax
from jax.experimental import pallas as pl
from jax.experimental.pallas import tpu as pltpu
from jax.experimental.pallas import tpu_sc as plsc

NUM_EXPERTS = 64
TOP_K = 8
HIDDEN = 4096
BATCH = 2
SEQ = 4096
TOKENS = BATCH * SEQ
TILE_T = 1024

NC = 2
NS = 16
NW = NC * NS
T_SUB = TOKENS // NW
LANES = 16


def _probs_body(x_ref, w_ref, probsT_ref, psum_ref, acc_ref):
    i = pl.program_id(0)

    @pl.when(i == 0)
    def _init():
        acc_ref[...] = jnp.zeros_like(acc_ref)

    x = x_ref[...]
    w = w_ref[...]
    logitsT = jax.lax.dot_general(
        w, x, (((1,), (1,)), ((), ())),
        preferred_element_type=jnp.float32)
    m = jnp.max(logitsT, axis=0, keepdims=True)
    e = jnp.exp(logitsT - m)
    s = jnp.sum(e, axis=0, keepdims=True)
    probsT = e / s
    probsT_ref[...] = probsT
    acc_ref[...] += jnp.sum(probsT, axis=1, keepdims=True)

    @pl.when(i == pl.num_programs(0) - 1)
    def _fin():
        psum_ref[...] = acc_ref[...]


def _tc_probs(x2d, W):
    grid = TOKENS // TILE_T
    return pl.pallas_call(
        _probs_body,
        grid=(grid,),
        in_specs=[
            pl.BlockSpec((TILE_T, HIDDEN), lambda i: (i, 0)),
            pl.BlockSpec((NUM_EXPERTS, HIDDEN), lambda i: (0, 0)),
        ],
        out_specs=[
            pl.BlockSpec((NUM_EXPERTS, TILE_T), lambda i: (0, i)),
            pl.BlockSpec((NUM_EXPERTS, 1), lambda i: (0, 0)),
        ],
        out_shape=[
            jax.ShapeDtypeStruct((NUM_EXPERTS, TOKENS), jnp.float32),
            jax.ShapeDtypeStruct((NUM_EXPERTS, 1), jnp.float32),
        ],
        scratch_shapes=[pltpu.VMEM((NUM_EXPERTS, 1), jnp.float32)],
        compiler_params=pltpu.CompilerParams(
            dimension_semantics=("arbitrary",),
        ),
    )(x2d, W)


def _sc_topk_body(probsT_hbm, vals_hbm, idx_hbm, cnt_hbm,
                  stripe, vals_v, idx_v, cnt_v, sem):
    wid = lax.axis_index("s") * NC + lax.axis_index("c")
    base = wid * T_SUB
    pltpu.sync_copy(probsT_hbm.at[:, pl.ds(base, T_SUB)], stripe)

    for j in range(NUM_EXPERTS // LANES):
        cnt_v[pl.ds(j * LANES, LANES)] = jnp.zeros((LANES,), jnp.float32)

    lane = lax.iota(jnp.int32, LANES)
    onesf = jnp.ones((LANES,), jnp.float32)
    neg1 = jnp.full((LANES,), -1.0, jnp.float32)
    zeroi = jnp.zeros((LANES,), jnp.int32)

    def group(g, _):
        m = [neg1] * TOP_K
        mi = [zeroi] * TOP_K
        for e in range(NUM_EXPERTS):
            v = stripe[e, pl.ds(g * LANES, LANES)]
            ei = jnp.full((LANES,), e, jnp.int32)
            c = [v > m[j] for j in range(TOP_K)]
            for j in range(TOP_K - 1, 0, -1):
                m[j] = jnp.where(c[j], jnp.where(c[j - 1], m[j - 1], v), m[j])
                mi[j] = jnp.where(c[j], jnp.where(c[j - 1], mi[j - 1], ei), mi[j])
            m[0] = jnp.where(c[0], v, m[0])
            mi[0] = jnp.where(c[0], ei, mi[0])
        s = m[0]
        for k in range(1, TOP_K):
            s = s + m[k]
        tok8 = (g * LANES + lane) * TOP_K
        for k in range(TOP_K):
            kk = tok8 + k
            plsc.store_scatter(vals_v, [kk], m[k] / s)
            plsc.store_scatter(idx_v, [kk], mi[k])
            plsc.addupdate_scatter(cnt_v, [mi[k]], onesf)
        return 0

    lax.fori_loop(0, T_SUB // LANES, group, 0)

    pltpu.sync_copy(vals_v, vals_hbm.at[pl.ds(base * TOP_K, T_SUB * TOP_K)])
    pltpu.sync_copy(idx_v, idx_hbm.at[pl.ds(base * TOP_K, T_SUB * TOP_K)])
    pltpu.sync_copy(cnt_v, cnt_hbm.at[wid])


def _sc_topk(probsT):
    mesh = plsc.VectorSubcoreMesh(
        core_axis_name="c", subcore_axis_name="s",
        num_cores=NC, num_subcores=NS)
    f = pl.kernel(
        _sc_topk_body,
        out_type=[
            jax.ShapeDtypeStruct((TOKENS * TOP_K,), jnp.float32),
            jax.ShapeDtypeStruct((TOKENS * TOP_K,), jnp.int32),
            jax.ShapeDtypeStruct((NW, NUM_EXPERTS), jnp.float32),
        ],
        mesh=mesh,
        scratch_types=[
            pltpu.VMEM((NUM_EXPERTS, T_SUB), jnp.float32),
            pltpu.VMEM((T_SUB * TOP_K,), jnp.float32),
            pltpu.VMEM((T_SUB * TOP_K,), jnp.int32),
            pltpu.VMEM((NUM_EXPERTS,), jnp.float32),
            pltpu.SemaphoreType.DMA,
        ],
        compiler_params=pltpu.CompilerParams(needs_layout_passes=False),
    )
    return f(probsT)


def _aux_body(cnt_ref, psum_ref, aux_ref):
    cnt = cnt_ref[...]
    total = jnp.sum(cnt, axis=0, keepdims=True)
    prod = jax.lax.dot_general(
        total, psum_ref[...], (((1,), (0,)), ((), ())),
        preferred_element_type=jnp.float32)
    aux_ref[...] = prod * (jnp.float32(NUM_EXPERTS)
                           / jnp.float32(BATCH) / jnp.float32(TOKENS))


def _tc_aux(cnt, psum):
    return pl.pallas_call(
        _aux_body,
        out_shape=jax.ShapeDtypeStruct((1, 1), jnp.float32),
    )(cnt, psum)


def kernel(x, W):
    x2d = x.reshape(TOKENS, HIDDEN)
    probsT, psum = _tc_probs(x2d, W)
    vals, idxs, cnt = _sc_topk(probsT)
    aux = _tc_aux(cnt, psum)
    return (vals.reshape(BATCH, SEQ, TOP_K),
            idxs.reshape(BATCH, SEQ, TOP_K),
            aux[0, 0])
